# baseline (speedup 1.0000x reference)
import dataclasses

import jax
import jax.numpy as jnp
from jax.experimental import pallas as pl
from jax.experimental.pallas import tpu as pltpu
from jax.experimental.pallas import tpu_sc as plsc

B = 2
Q = 150
C = 150
K = 100
KP = 128
H = 384
W = 384
BH = 16
NBLK = H // BH
BIGI = 2147483647
HQ = 80
QP = 160


def _topk_body(logits_ref, probs_t_ref, vals_ref, labels_ref, q_ref, oh_ref):
    lg = logits_ref[...]
    m = jnp.max(lg, axis=-1, keepdims=True)
    e = jnp.exp(lg - m)
    p = e / jnp.sum(e, axis=-1, keepdims=True)
    scores = p[:, :, :C]
    probs_t_ref[...] = jnp.swapaxes(scores, 1, 2)

    qi3 = jax.lax.broadcasted_iota(jnp.int32, (1, Q, 1), 1)
    ci3 = jax.lax.broadcasted_iota(jnp.int32, (1, 1, C), 2)
    li = jax.lax.broadcasted_iota(jnp.int32, (1, 1, KP), 2)

    colmax0 = jnp.max(scores, axis=1, keepdims=True)
    bestq0 = jnp.min(jnp.where(scores == colmax0, qi3, BIGI),
                     axis=1, keepdims=True)

    def step(i, carry):
        x, colmax, bestq, acc_v, acc_i = carry
        mv = jnp.max(colmax, axis=2, keepdims=True)
        fcand = jnp.where(colmax == mv, bestq * C + ci3, BIGI)
        fsel = jnp.min(fcand, axis=2, keepdims=True)
        acc_v = jnp.where(li == i, mv, acc_v)
        acc_i = jnp.where(li == i, fsel, acc_i)
        cc = fsel % C
        bq = fsel // C
        x = jnp.where((qi3 == bq) & (ci3 == cc), -1.0, x)
        colsel = ci3 == cc
        xm = jnp.where(colsel, x, -1.0)
        colcc = jnp.max(xm, axis=1, keepdims=True)
        bqcc = jnp.min(jnp.where(xm == colcc, qi3, BIGI),
                       axis=1, keepdims=True)
        colmax = jnp.where(colsel, colcc, colmax)
        bestq = jnp.where(colsel, bqcc, bestq)
        return x, colmax, bestq, acc_v, acc_i

    acc_v0 = jnp.zeros((B, 1, KP), jnp.float32)
    acc_i0 = jnp.zeros((B, 1, KP), jnp.int32)
    _, _, _, acc_v, acc_i = jax.lax.fori_loop(
        0, K, step, (scores, colmax0, bestq0, acc_v0, acc_i0))

    vals_ref[...] = acc_v[:, 0, :]
    labels_ref[...] = (acc_i % C)[:, 0, :]
    qv = acc_i // C
    q_ref[...] = qv[:, 0, :]
    kmask = li < K
    qi = jax.lax.broadcasted_iota(jnp.int32, (1, Q, 1), 1)
    oht = jnp.where((qv == qi) & kmask, 1.0, 0.0)
    oh_ref[...] = jnp.swapaxes(oht, 1, 2)


def _mega_body(masks_ref, pt_ref, oh_ref,
               sem_ref, inst_ref, rt_ref, acc_n, acc_d):
    n = pl.program_id(1)
    xb = masks_ref[0].astype(jnp.bfloat16).reshape(Q, BH * W)
    sig = jax.nn.sigmoid(xb)
    one = jnp.ones((), jnp.bfloat16)
    ind = jnp.where(xb > 0, one, jnp.zeros((), jnp.bfloat16))
    sem2 = jnp.dot(pt_ref[0].astype(jnp.bfloat16), sig,
                   preferred_element_type=jnp.float32)
    sem_ref[0] = sem2.reshape(C, BH, W)
    io = jnp.dot(oh_ref[0].astype(jnp.bfloat16), ind,
                 preferred_element_type=jnp.float32)
    inst_ref[0] = io[:K].reshape(K, BH, W)
    masked = jnp.where(xb > 0, sig, jnp.zeros((), jnp.bfloat16))
    pn = jnp.sum(masked, axis=1, keepdims=True, dtype=jnp.float32)
    pd = jnp.sum(ind, axis=1, keepdims=True, dtype=jnp.float32)

    @pl.when(n == 0)
    def _():
        acc_n[...] = pn
        acc_d[...] = pd

    @pl.when(n > 0)
    def _():
        acc_n[...] += pn
        acc_d[...] += pd

    @pl.when(n == NBLK - 1)
    def _():
        ratio = acc_n[...] / (acc_d[...] + 1e-6)
        row = jnp.swapaxes(ratio, 0, 1)
        rt_ref[0] = jnp.concatenate(
            [row, jnp.zeros((1, QP - Q), jnp.float32)], axis=1)


def _sc_final(vals2, q2, ratiot):
    vec_mesh = plsc.VectorSubcoreMesh(core_axis_name="c",
                                      subcore_axis_name="s")
    cp = pltpu.CompilerParams()
    if "needs_layout_passes" in pltpu.CompilerParams.__dataclass_fields__:
        cp = dataclasses.replace(cp, needs_layout_passes=False)

    @pl.kernel(
        out_type=jax.ShapeDtypeStruct((B, KP), jnp.float32),
        mesh=vec_mesh,
        compiler_params=cp,
        scratch_types=[
            pltpu.VMEM((16,), jnp.float32),
            pltpu.VMEM((16,), jnp.int32),
            pltpu.VMEM((QP,), jnp.float32),
            pltpu.VMEM((16,), jnp.float32),
            pltpu.SemaphoreType.DMA,
            pltpu.SemaphoreType.DMA,
            pltpu.SemaphoreType.DMA,
        ],
    )
    def sc_kernel(vals_hbm, q_hbm, rt_hbm, o_hbm, sv, sq, sr, so,
                  sem1, sem2, sem3):
        b = jax.lax.axis_index("c")
        s = jax.lax.axis_index("s")

        @pl.when(s < 8)
        def _():
            off = s * 16
            c1 = pltpu.async_copy(vals_hbm.at[b, pl.ds(off, 16)], sv, sem1)
            c2 = pltpu.async_copy(q_hbm.at[b, pl.ds(off, 16)], sq, sem2)
            c3 = pltpu.async_copy(rt_hbm.at[b, 0], sr, sem3)
            c1.wait()
            c2.wait()
            c3.wait()
            gr = plsc.load_gather(sr, (sq[...],))
            so[...] = sv[...] * gr
            pltpu.sync_copy(so, o_hbm.at[b, pl.ds(off, 16)])

    return sc_kernel(vals2, q2, ratiot)


def kernel(pred_logits, pred_masks):
    probs_t, vals, labels, qidx, oh = pl.pallas_call(
        _topk_body,
        out_shape=(
            jax.ShapeDtypeStruct((B, C, Q), jnp.float32),
            jax.ShapeDtypeStruct((B, KP), jnp.float32),
            jax.ShapeDtypeStruct((B, KP), jnp.int32),
            jax.ShapeDtypeStruct((B, KP), jnp.int32),
            jax.ShapeDtypeStruct((B, KP, Q), jnp.float32),
        ),
    )(pred_logits)

    sem, inst, rt = pl.pallas_call(
        _mega_body,
        grid=(B, NBLK),
        in_specs=[
            pl.BlockSpec((1, Q, BH, W), lambda b, n: (b, 0, n, 0)),
            pl.BlockSpec((1, C, Q), lambda b, n: (b, 0, 0)),
            pl.BlockSpec((1, KP, Q), lambda b, n: (b, 0, 0)),
        ],
        out_specs=(
            pl.BlockSpec((1, C, BH, W), lambda b, n: (b, 0, n, 0)),
            pl.BlockSpec((1, K, BH, W), lambda b, n: (b, 0, n, 0)),
            pl.BlockSpec((1, 1, QP), lambda b, n: (b, 0, 0)),
        ),
        out_shape=(
            jax.ShapeDtypeStruct((B, C, H, W), jnp.float32),
            jax.ShapeDtypeStruct((B, K, H, W), jnp.float32),
            jax.ShapeDtypeStruct((B, 1, QP), jnp.float32),
        ),
        scratch_shapes=[
            pltpu.VMEM((Q, 1), jnp.float32),
            pltpu.VMEM((Q, 1), jnp.float32),
        ],
        compiler_params=pltpu.CompilerParams(
            dimension_semantics=("arbitrary", "arbitrary")),
    )(pred_masks, probs_t, oh)

    fin = _sc_final(vals, qidx, rt)

    final_scores = fin[:, :K]
    labels_out = labels[:, :K]
    return sem, inst, final_scores, labels_out


# speedup vs baseline: 1.0236x; 1.0236x over previous
import dataclasses

import jax
import jax.numpy as jnp
from jax.experimental import pallas as pl
from jax.experimental.pallas import tpu as pltpu
from jax.experimental.pallas import tpu_sc as plsc

B = 2
Q = 150
C = 150
K = 100
KP = 128
H = 384
W = 384
BH = 32
NBLK = H // BH
BIGI = 2147483647
HQ = 80
QP = 160


def _topk_body(logits_ref, probs_t_ref, vals_ref, labels_ref, q_ref, oh_ref):
    lg = logits_ref[...]
    m = jnp.max(lg, axis=-1, keepdims=True)
    e = jnp.exp(lg - m)
    p = e / jnp.sum(e, axis=-1, keepdims=True)
    scores = p[:, :, :C]
    probs_t_ref[...] = jnp.swapaxes(scores, 1, 2)

    qi3 = jax.lax.broadcasted_iota(jnp.int32, (1, Q, 1), 1)
    ci3 = jax.lax.broadcasted_iota(jnp.int32, (1, 1, C), 2)
    li = jax.lax.broadcasted_iota(jnp.int32, (1, 1, KP), 2)

    colmax0 = jnp.max(scores, axis=1, keepdims=True)
    bestq0 = jnp.min(jnp.where(scores == colmax0, qi3, BIGI),
                     axis=1, keepdims=True)

    def step(i, carry):
        x, colmax, bestq, acc_v, acc_i = carry
        mv = jnp.max(colmax, axis=2, keepdims=True)
        fcand = jnp.where(colmax == mv, bestq * C + ci3, BIGI)
        fsel = jnp.min(fcand, axis=2, keepdims=True)
        acc_v = jnp.where(li == i, mv, acc_v)
        acc_i = jnp.where(li == i, fsel, acc_i)
        cc = fsel % C
        bq = fsel // C
        x = jnp.where((qi3 == bq) & (ci3 == cc), -1.0, x)
        colsel = ci3 == cc
        xm = jnp.where(colsel, x, -1.0)
        colcc = jnp.max(xm, axis=1, keepdims=True)
        bqcc = jnp.min(jnp.where(xm == colcc, qi3, BIGI),
                       axis=1, keepdims=True)
        colmax = jnp.where(colsel, colcc, colmax)
        bestq = jnp.where(colsel, bqcc, bestq)
        return x, colmax, bestq, acc_v, acc_i

    acc_v0 = jnp.zeros((B, 1, KP), jnp.float32)
    acc_i0 = jnp.zeros((B, 1, KP), jnp.int32)
    _, _, _, acc_v, acc_i = jax.lax.fori_loop(
        0, K, step, (scores, colmax0, bestq0, acc_v0, acc_i0))

    vals_ref[...] = acc_v[:, 0, :]
    labels_ref[...] = (acc_i % C)[:, 0, :]
    qv = acc_i // C
    q_ref[...] = qv[:, 0, :]
    kmask = li < K
    qi = jax.lax.broadcasted_iota(jnp.int32, (1, Q, 1), 1)
    oht = jnp.where((qv == qi) & kmask, 1.0, 0.0)
    oh_ref[...] = jnp.swapaxes(oht, 1, 2)


def _mega_body(masks_ref, pt_ref, oh_ref,
               sem_ref, inst_ref, rt_ref, acc_n, acc_d):
    n = pl.program_id(1)
    xb = masks_ref[0].astype(jnp.bfloat16).reshape(Q, BH * W)
    sig = jax.nn.sigmoid(xb)
    one = jnp.ones((), jnp.bfloat16)
    ind = jnp.where(xb > 0, one, jnp.zeros((), jnp.bfloat16))
    sem2 = jnp.dot(pt_ref[0].astype(jnp.bfloat16), sig,
                   preferred_element_type=jnp.float32)
    sem_ref[0] = sem2.reshape(C, BH, W)
    io = jnp.dot(oh_ref[0].astype(jnp.bfloat16), ind,
                 preferred_element_type=jnp.float32)
    inst_ref[0] = io[:K].reshape(K, BH, W)
    masked = jnp.where(xb > 0, sig, jnp.zeros((), jnp.bfloat16))
    pn = jnp.sum(masked, axis=1, keepdims=True, dtype=jnp.float32)
    pd = jnp.sum(ind, axis=1, keepdims=True, dtype=jnp.float32)

    @pl.when(n == 0)
    def _():
        acc_n[...] = pn
        acc_d[...] = pd

    @pl.when(n > 0)
    def _():
        acc_n[...] += pn
        acc_d[...] += pd

    @pl.when(n == NBLK - 1)
    def _():
        ratio = acc_n[...] / (acc_d[...] + 1e-6)
        row = jnp.swapaxes(ratio, 0, 1)
        rt_ref[0] = jnp.concatenate(
            [row, jnp.zeros((1, QP - Q), jnp.float32)], axis=1)


def _sc_final(vals2, q2, ratiot):
    vec_mesh = plsc.VectorSubcoreMesh(core_axis_name="c",
                                      subcore_axis_name="s")
    cp = pltpu.CompilerParams()
    if "needs_layout_passes" in pltpu.CompilerParams.__dataclass_fields__:
        cp = dataclasses.replace(cp, needs_layout_passes=False)

    @pl.kernel(
        out_type=jax.ShapeDtypeStruct((B, KP), jnp.float32),
        mesh=vec_mesh,
        compiler_params=cp,
        scratch_types=[
            pltpu.VMEM((16,), jnp.float32),
            pltpu.VMEM((16,), jnp.int32),
            pltpu.VMEM((QP,), jnp.float32),
            pltpu.VMEM((16,), jnp.float32),
        ],
    )
    def sc_kernel(vals_hbm, q_hbm, rt_hbm, o_hbm, sv, sq, sr, so):
        b = jax.lax.axis_index("c")
        s = jax.lax.axis_index("s")

        @pl.when(s < 8)
        def _():
            off = s * 16
            pltpu.sync_copy(vals_hbm.at[b, pl.ds(off, 16)], sv)
            pltpu.sync_copy(q_hbm.at[b, pl.ds(off, 16)], sq)
            pltpu.sync_copy(rt_hbm.at[b, 0], sr)
            gr = plsc.load_gather(sr, (sq[...],))
            so[...] = sv[...] * gr
            pltpu.sync_copy(so, o_hbm.at[b, pl.ds(off, 16)])

    return sc_kernel(vals2, q2, ratiot)


def kernel(pred_logits, pred_masks):
    probs_t, vals, labels, qidx, oh = pl.pallas_call(
        _topk_body,
        out_shape=(
            jax.ShapeDtypeStruct((B, C, Q), jnp.float32),
            jax.ShapeDtypeStruct((B, KP), jnp.float32),
            jax.ShapeDtypeStruct((B, KP), jnp.int32),
            jax.ShapeDtypeStruct((B, KP), jnp.int32),
            jax.ShapeDtypeStruct((B, KP, Q), jnp.float32),
        ),
    )(pred_logits)

    sem, inst, rt = pl.pallas_call(
        _mega_body,
        grid=(B, NBLK),
        in_specs=[
            pl.BlockSpec((1, Q, BH, W), lambda b, n: (b, 0, n, 0)),
            pl.BlockSpec((1, C, Q), lambda b, n: (b, 0, 0)),
            pl.BlockSpec((1, KP, Q), lambda b, n: (b, 0, 0)),
        ],
        out_specs=(
            pl.BlockSpec((1, C, BH, W), lambda b, n: (b, 0, n, 0)),
            pl.BlockSpec((1, K, BH, W), lambda b, n: (b, 0, n, 0)),
            pl.BlockSpec((1, 1, QP), lambda b, n: (b, 0, 0)),
        ),
        out_shape=(
            jax.ShapeDtypeStruct((B, C, H, W), jnp.float32),
            jax.ShapeDtypeStruct((B, K, H, W), jnp.float32),
            jax.ShapeDtypeStruct((B, 1, QP), jnp.float32),
        ),
        scratch_shapes=[
            pltpu.VMEM((Q, 1), jnp.float32),
            pltpu.VMEM((Q, 1), jnp.float32),
        ],
        compiler_params=pltpu.CompilerParams(
            dimension_semantics=("arbitrary", "arbitrary")),
    )(pred_masks, probs_t, oh)

    fin = _sc_final(vals, qidx, rt)

    final_scores = fin[:, :K]
    labels_out = labels[:, :K]
    return sem, inst, final_scores, labels_out
